# baseline (speedup 1.0000x reference)
import functools

import jax
import jax.numpy as jnp
from jax import lax
from jax.experimental import pallas as pl
from jax.experimental.pallas import tpu as pltpu
from jax.experimental.pallas import tpu_sc as plsc

N_POINTS = 100000
CIN = 64
COUT = 64
NK = 16
BEV_H = 256
BEV_W = 256
BATCH = 4
NROWS = BATCH * BEV_H * BEV_W

NC = 2
NS = 16

N_PAD = 114688
P_TILE = N_PAD // NS

CHUNK = 32768
NCHUNKS = 8
NCPC = NCHUNKS // NC
ROWS_PER_TILE = CHUNK // NS

BINCAP = 16384
BIN_TILE = BINCAP // NS
TRASH = NCHUNKS * BINCAP
BINS_TOTAL = TRASH + 8
PID_DUMMY = 110592
PACKED_DUMMY = PID_DUMMY

MM_BLK = 1024
MM_GRID = N_PAD // MM_BLK
MM_LAST = (N_POINTS - 1) // MM_BLK


def _matmul_tc(features, kidx, x0, x1, b, kern2, lt):
    def body(feat_ref, kidx_ref, x0_ref, x1_ref, b_ref, kern_ref, lt_ref,
             sf_ref, dest_ref, packed_ref, carry_ref):
        i = pl.program_id(0)

        @pl.when(i == 0)
        def _():
            carry_ref[...] = jnp.zeros((1, NCHUNKS), jnp.float32)

        feat = feat_ref[...].astype(jnp.bfloat16)
        kidx = kidx_ref[...].reshape(MM_BLK, 1)
        ks = lax.broadcasted_iota(jnp.int32, (1, NK), 1)
        oh = (kidx == ks).astype(jnp.bfloat16)
        r1 = lax.broadcasted_iota(jnp.int32, (NK, NK * CIN), 0)
        r2 = lax.broadcasted_iota(jnp.int32, (NK, NK * CIN), 1) // CIN
        expander = (r1 == r2).astype(jnp.bfloat16)
        ohbig = jnp.dot(oh, expander,
                        preferred_element_type=jnp.float32
                        ).astype(jnp.bfloat16)
        big = jnp.concatenate([feat] * NK, axis=1) * ohbig
        res = jnp.dot(big, kern_ref[...], preferred_element_type=jnp.float32)
        rowid2 = (i * MM_BLK
                  + lax.broadcasted_iota(jnp.int32, (MM_BLK, 1), 0))
        res = jnp.where(rowid2 < N_POINTS, res, 0.0)
        sf_ref[...] = res.astype(jnp.bfloat16)

        rowid = i * MM_BLK + lax.broadcasted_iota(jnp.int32, (MM_BLK,), 0)
        flat = (b_ref[...] * (BEV_H * BEV_W)
                + x0_ref[...] * BEV_W + x1_ref[...])
        flat = jnp.where(rowid < N_POINTS, flat, -1)
        ch2 = lax.shift_right_arithmetic(flat, 15).reshape(MM_BLK, 1)
        cs = lax.broadcasted_iota(jnp.int32, (1, NCHUNKS), 1)
        oh8 = (ch2 == cs).astype(jnp.float32)

        pcount = jnp.dot(lt_ref[...], oh8.astype(jnp.bfloat16),
                         preferred_element_type=jnp.float32)
        base = carry_ref[...]
        rank = jnp.sum((pcount + base) * oh8, axis=1, keepdims=True) - 1.0
        carry_ref[...] = base + jnp.sum(oh8, axis=0, keepdims=True)

        ranki = jnp.minimum(rank.astype(jnp.int32), BINCAP - 1)
        dest2 = ch2 * BINCAP + ranki
        dest2 = jnp.where(ch2 >= 0, dest2, TRASH)
        dest_ref[...] = dest2.reshape(MM_BLK)
        packed_ref[...] = jnp.bitwise_or(
            lax.shift_left(jnp.bitwise_and(flat, CHUNK - 1), 17), rowid)

    def pt_map(i):
        return (jnp.minimum(i, MM_LAST), 0)

    int_spec = pl.BlockSpec((MM_BLK,), lambda i: (jnp.minimum(i, MM_LAST),))
    out_1d = pl.BlockSpec((MM_BLK,), lambda i: (i,))
    sf, dest, packed = pl.pallas_call(
        body,
        grid=(MM_GRID,),
        in_specs=[
            pl.BlockSpec((MM_BLK, CIN), pt_map),
            int_spec, int_spec, int_spec, int_spec,
            pl.BlockSpec((NK * CIN, COUT), lambda i: (0, 0)),
            pl.BlockSpec((MM_BLK, MM_BLK), lambda i: (0, 0)),
        ],
        out_specs=[
            pl.BlockSpec((MM_BLK, COUT), lambda i: (i, 0)),
            out_1d, out_1d,
        ],
        out_shape=[
            jax.ShapeDtypeStruct((N_PAD, COUT), jnp.bfloat16),
            jax.ShapeDtypeStruct((N_PAD,), jnp.int32),
            jax.ShapeDtypeStruct((N_PAD,), jnp.int32),
        ],
        scratch_shapes=[pltpu.VMEM((1, NCHUNKS), jnp.float32)],
    )(features, kidx, x0, x1, b, kern2, lt)
    return sf, dest, packed


def _scatter_sc(rows, dest, packed, zeros_hbm):
    mesh = plsc.VectorSubcoreMesh(
        core_axis_name="c", subcore_axis_name="s",
        num_cores=NC, num_subcores=NS)

    @functools.partial(
        pl.kernel,
        out_type=[
            jax.ShapeDtypeStruct((NCHUNKS * CHUNK, COUT), jnp.bfloat16),
            jax.ShapeDtypeStruct((BINS_TOTAL,), jnp.int32),
        ],
        mesh=mesh,
        compiler_params=pltpu.CompilerParams(use_tc_tiling_on_sc=False),
        scratch_types=[
            pltpu.VMEM((P_TILE,), jnp.int32),
            pltpu.VMEM((P_TILE,), jnp.int32),
            pltpu.VMEM((512,), jnp.int32),
            pltpu.VMEM((512,), jnp.int32),
            pltpu.VMEM((BIN_TILE,), jnp.int32),
            pltpu.VMEM((BIN_TILE,), jnp.int32),
            pltpu.VMEM((BIN_TILE,), jnp.int32),
            pltpu.VMEM((BIN_TILE, COUT), jnp.bfloat16),
            pltpu.VMEM_SHARED((CHUNK, COUT), jnp.bfloat16),
        ],
    )
    def k(rows_hbm, dest_hbm, packed_hbm, zsrc_hbm, bev_hbm, bins_hbm,
          dest_v, packed_v, destr, packr, bin_v, pidr, lidxr, rows_v, acc):
        cid = lax.axis_index("c")
        sid = lax.axis_index("s")
        base_pt = sid * P_TILE
        lo = cid * (NCPC * BINCAP)
        hi = lo + NCPC * BINCAP

        pltpu.sync_copy(dest_hbm.at[pl.ds(base_pt, P_TILE)], dest_v)
        pltpu.sync_copy(packed_hbm.at[pl.ds(base_pt, P_TILE)], packed_v)

        lane = lax.broadcasted_iota(jnp.int32, (16,), 0)

        def dfill(i2, c):
            packr[pl.ds(i2 * 16, 16)] = lane * 0 + PACKED_DUMMY
            return c
        lax.fori_loop(0, 512 // 16, dfill, 0)
        for w in range(NCPC * BINCAP // (512 * NS)):
            pltpu.sync_copy(
                packr,
                bins_hbm.at[pl.ds(lo + (sid * 8 + w) * 512, 512)])
        plsc.subcore_barrier()

        for w in range(P_TILE // 512):
            def wcopy(i2, c, _w=w):
                d = dest_v[pl.ds(_w * 512 + i2 * 16, 16)]
                mine = (d >= lo) & (d < hi)
                destr[pl.ds(i2 * 16, 16)] = jnp.where(mine, d, TRASH)
                packr[pl.ds(i2 * 16, 16)] = (
                    packed_v[pl.ds(_w * 512 + i2 * 16, 16)])
                return c
            lax.fori_loop(0, 512 // 16, wcopy, 0)
            pltpu.sync_copy(packr, bins_hbm.at[destr])
        plsc.subcore_barrier()

        def per_chunk(p, carry):
            chunk_id = cid * NCPC + p
            chunk_base = chunk_id * CHUNK

            pltpu.sync_copy(
                zsrc_hbm,
                acc.at[pl.ds(sid * ROWS_PER_TILE, ROWS_PER_TILE), :])
            pltpu.sync_copy(
                bins_hbm.at[pl.ds(chunk_id * BINCAP + sid * BIN_TILE,
                                  BIN_TILE)],
                bin_v)

            def unp(i2, c):
                pk = bin_v[pl.ds(i2 * 16, 16)]
                pidr[pl.ds(i2 * 16, 16)] = jnp.bitwise_and(pk, 0x1FFFF)
                lidxr[pl.ds(i2 * 16, 16)] = jnp.bitwise_and(
                    lax.shift_right_logical(pk, 17), CHUNK - 1)
                return c
            lax.fori_loop(0, BIN_TILE // 16, unp, 0)
            plsc.subcore_barrier()

            pltpu.sync_copy(rows_hbm.at[pidr], rows_v)
            pltpu.sync_copy(rows_v, acc.at[lidxr], add=True)
            plsc.subcore_barrier()

            pltpu.sync_copy(
                acc.at[pl.ds(sid * ROWS_PER_TILE, ROWS_PER_TILE), :],
                bev_hbm.at[pl.ds(chunk_base + sid * ROWS_PER_TILE,
                                 ROWS_PER_TILE), :])
            plsc.subcore_barrier()
            return carry

        lax.fori_loop(0, NCPC, per_chunk, 0)

    return k(rows, dest, packed, zeros_hbm)


def _transpose_tc(bev):
    HW = BEV_H * BEV_W
    TB = 4096
    SH = TB // BEV_W

    def body(in_ref, out_ref):
        x = in_ref[...].astype(jnp.float32).T
        out_ref[...] = x.reshape(1, COUT, SH, BEV_W)

    return pl.pallas_call(
        body,
        grid=(BATCH, HW // TB),
        in_specs=[pl.BlockSpec((TB, COUT),
                               lambda b, i: (b * (HW // TB) + i, 0))],
        out_specs=pl.BlockSpec((1, COUT, SH, BEV_W),
                               lambda b, i: (b, 0, i, 0)),
        out_shape=jax.ShapeDtypeStruct((BATCH, COUT, BEV_H, BEV_W),
                                       jnp.float32),
    )(bev)


def kernel(features, coords_d0, coords_d1, coords_d2, coords_b, stride,
           kernel):
    kidx = (coords_d1 // stride).astype(jnp.int32)
    x0 = (coords_d0 // stride).astype(jnp.int32)
    x1 = (coords_d2 // stride).astype(jnp.int32)
    b = coords_b.astype(jnp.int32)
    kern2 = kernel.reshape(NK * CIN, COUT).astype(jnp.bfloat16)
    ar = jnp.arange(MM_BLK, dtype=jnp.int32)
    lt = (ar[:, None] >= ar[None, :]).astype(jnp.bfloat16)

    sf, dest, packed = _matmul_tc(features, kidx, x0, x1, b, kern2, lt)

    zeros_hbm = jnp.zeros((ROWS_PER_TILE, COUT), jnp.bfloat16)
    bev, _ = _scatter_sc(sf, dest, packed, zeros_hbm)

    return _transpose_tc(bev)


# speedup vs baseline: 23.0769x; 23.0769x over previous
import functools

import jax
import jax.numpy as jnp
from jax import lax
from jax.experimental import pallas as pl
from jax.experimental.pallas import tpu as pltpu
from jax.experimental.pallas import tpu_sc as plsc

N_POINTS = 100000
CIN = 64
COUT = 64
NK = 16
BEV_H = 256
BEV_W = 256
BATCH = 4
NROWS = BATCH * BEV_H * BEV_W

NC = 2
NS = 16

N_PAD = 114688
P_TILE = N_PAD // NS
BLK = 512
NBLK = P_TILE // BLK

CHUNK = 32768
NCHUNKS = 8
NCPC = NCHUNKS // NC
ROWS_PER_TILE = CHUNK // NS

MM_BLK = 1024
MM_GRID = N_PAD // MM_BLK
MM_LAST = (N_POINTS - 1) // MM_BLK


def _matmul_tc(features, kidx, x0, x1, b, kern2):
    def body(feat_ref, kidx_ref, x0_ref, x1_ref, b_ref, kern_ref,
             sf_ref, flat_ref):
        i = pl.program_id(0)
        feat = feat_ref[...].astype(jnp.bfloat16)
        kidx = kidx_ref[...].reshape(MM_BLK, 1)
        ks = lax.broadcasted_iota(jnp.int32, (1, NK), 1)
        oh = (kidx == ks).astype(jnp.bfloat16)
        r1 = lax.broadcasted_iota(jnp.int32, (NK, NK * CIN), 0)
        r2 = lax.broadcasted_iota(jnp.int32, (NK, NK * CIN), 1) // CIN
        expander = (r1 == r2).astype(jnp.bfloat16)
        ohbig = jnp.dot(oh, expander,
                        preferred_element_type=jnp.float32
                        ).astype(jnp.bfloat16)
        big = jnp.concatenate([feat] * NK, axis=1) * ohbig
        sf_ref[...] = jnp.dot(big, kern_ref[...],
                              preferred_element_type=jnp.float32
                              ).astype(jnp.bfloat16)
        rowid = i * MM_BLK + lax.broadcasted_iota(jnp.int32, (MM_BLK,), 0)
        flat = (b_ref[...] * (BEV_H * BEV_W)
                + x0_ref[...] * BEV_W + x1_ref[...])
        flat_ref[...] = jnp.where(rowid < N_POINTS, flat, -1)

    def pt_map(i):
        return (jnp.minimum(i, MM_LAST), 0)

    int_spec = pl.BlockSpec((MM_BLK,), lambda i: (jnp.minimum(i, MM_LAST),))
    sf, flat2 = pl.pallas_call(
        body,
        grid=(MM_GRID,),
        in_specs=[
            pl.BlockSpec((MM_BLK, CIN), pt_map),
            int_spec, int_spec, int_spec, int_spec,
            pl.BlockSpec((NK * CIN, COUT), lambda i: (0, 0)),
        ],
        out_specs=[
            pl.BlockSpec((MM_BLK, COUT), lambda i: (i, 0)),
            pl.BlockSpec((MM_BLK,), lambda i: (i,)),
        ],
        out_shape=[
            jax.ShapeDtypeStruct((N_PAD, COUT), jnp.bfloat16),
            jax.ShapeDtypeStruct((N_PAD,), jnp.int32),
        ],
    )(features, kidx, x0, x1, b, kern2)
    return sf, flat2


def _scatter_sc(rows, flat, zeros_hbm):
    mesh = plsc.VectorSubcoreMesh(
        core_axis_name="c", subcore_axis_name="s",
        num_cores=NC, num_subcores=NS)

    @functools.partial(
        pl.kernel,
        out_type=jax.ShapeDtypeStruct((NCHUNKS * CHUNK, COUT), jnp.bfloat16),
        mesh=mesh,
        compiler_params=pltpu.CompilerParams(use_tc_tiling_on_sc=False),
        scratch_types=[
            pltpu.VMEM((P_TILE,), jnp.int32),
            pltpu.VMEM((BLK, COUT), jnp.bfloat16),
            pltpu.VMEM((BLK, COUT), jnp.bfloat16),
            pltpu.VMEM((BLK,), jnp.int32),
            pltpu.VMEM((BLK,), jnp.int32),
            pltpu.VMEM_SHARED((CHUNK + 8, COUT), jnp.bfloat16),
            pltpu.SemaphoreType.DMA,
            pltpu.SemaphoreType.DMA,
            pltpu.SemaphoreType.DMA,
            pltpu.SemaphoreType.DMA,
        ],
    )
    def k(rows_hbm, flat_hbm, zsrc_hbm, bev_hbm, idx_v, rows0, rows1,
          lidx0, lidx1, acc, lsem0, lsem1, ssem0, ssem1):
        cid = lax.axis_index("c")
        sid = lax.axis_index("s")
        base_pt = sid * P_TILE
        rows_b = (rows0, rows1)
        lidx_b = (lidx0, lidx1)
        lsem_b = (lsem0, lsem1)
        ssem_b = (ssem0, ssem1)

        pltpu.sync_copy(flat_hbm.at[pl.ds(base_pt, P_TILE)], idx_v)

        def per_chunk(p, carry):
            chunk_id = cid * NCPC + p
            chunk_base = chunk_id * CHUNK

            pltpu.sync_copy(
                zsrc_hbm,
                acc.at[pl.ds(sid * ROWS_PER_TILE, ROWS_PER_TILE), :])
            plsc.subcore_barrier()

            loads = [None, None]
            scats = [None, None]
            loads[0] = pltpu.async_copy(
                rows_hbm.at[pl.ds(base_pt, BLK), :], rows0, lsem0)
            for bkt in range(NBLK):
                bb = bkt & 1
                nb = 1 - bb
                if bkt + 1 < NBLK:
                    if scats[nb] is not None:
                        scats[nb].wait()
                        scats[nb] = None
                    loads[nb] = pltpu.async_copy(
                        rows_hbm.at[pl.ds(base_pt + (bkt + 1) * BLK, BLK), :],
                        rows_b[nb], lsem_b[nb])
                loads[bb].wait()

                def sub_body(i2, c, _bkt=bkt, _bb=bb):
                    off = _bkt * BLK + i2 * 16
                    v = idx_v[pl.ds(off, 16)]
                    loc = v - chunk_base
                    ok = (loc >= 0) & (loc < CHUNK)
                    loc = jnp.where(ok, loc, CHUNK)
                    lidx_b[_bb][pl.ds(i2 * 16, 16)] = loc
                    return c
                lax.fori_loop(0, BLK // 16, sub_body, 0)
                if scats[bb] is not None:
                    scats[bb].wait()
                scats[bb] = pltpu.async_copy(
                    rows_b[bb], acc.at[lidx_b[bb]], ssem_b[bb], add=True)
            for s in scats:
                if s is not None:
                    s.wait()
            plsc.subcore_barrier()

            pltpu.sync_copy(
                acc.at[pl.ds(sid * ROWS_PER_TILE, ROWS_PER_TILE), :],
                bev_hbm.at[pl.ds(chunk_base + sid * ROWS_PER_TILE,
                                 ROWS_PER_TILE), :])
            plsc.subcore_barrier()
            return carry

        lax.fori_loop(0, NCPC, per_chunk, 0)

    return k(rows, flat, zeros_hbm)


def _transpose_tc(bev):
    HW = BEV_H * BEV_W
    TB = 4096
    SH = TB // BEV_W

    def body(in_ref, out_ref):
        x = in_ref[...].astype(jnp.float32).T
        out_ref[...] = x.reshape(1, COUT, SH, BEV_W)

    return pl.pallas_call(
        body,
        grid=(BATCH, HW // TB),
        in_specs=[pl.BlockSpec((TB, COUT),
                               lambda b, i: (b * (HW // TB) + i, 0))],
        out_specs=pl.BlockSpec((1, COUT, SH, BEV_W),
                               lambda b, i: (b, 0, i, 0)),
        out_shape=jax.ShapeDtypeStruct((BATCH, COUT, BEV_H, BEV_W),
                                       jnp.float32),
    )(bev)


def kernel(features, coords_d0, coords_d1, coords_d2, coords_b, stride,
           kernel):
    kidx = (coords_d1 // stride).astype(jnp.int32)
    x0 = (coords_d0 // stride).astype(jnp.int32)
    x1 = (coords_d2 // stride).astype(jnp.int32)
    b = coords_b.astype(jnp.int32)
    kern2 = kernel.reshape(NK * CIN, COUT).astype(jnp.bfloat16)

    sf, flat = _matmul_tc(features, kidx, x0, x1, b, kern2)

    zeros_hbm = jnp.zeros((ROWS_PER_TILE, COUT), jnp.bfloat16)
    bev = _scatter_sc(sf, flat, zeros_hbm)

    return _transpose_tc(bev)
